# baseline (speedup 1.0000x reference)
import functools

import jax
import jax.numpy as jnp
import numpy as np
from jax import lax
from jax.experimental import pallas as pl
from jax.experimental.pallas import tpu as pltpu
from jax.experimental.pallas import tpu_sc as plsc

H = 384
P = 8
NW = 32
QMAX = 70
BAND = 64
BWORDS = BAND * H
T2 = 250.0 * 250.0


def _window(x):
    sx = max(0, x - 20)
    ex = min(H, x + 19)
    return sx, ex


def _build_meta():
    qs = [(x, y) for x in range(0, H - P, P) for y in range(0, H - P, P)]
    n = len(qs)
    sizes = [70] + [69] * 31
    assert sum(sizes) == n
    mx = np.zeros((NW, 1, 96), dtype=np.int32)
    my = np.zeros((NW, 1, 96), dtype=np.int32)
    pos = 0
    for w, sz in enumerate(sizes):
        chunk = qs[pos:pos + sz]
        pos += sz
        sx0, _ = _window(chunk[0][0])
        _, ex1 = _window(chunk[-1][0])
        base = min(8 * (sx0 // 8), H - BAND)
        assert 0 <= base and ex1 - base <= BAND
        mx[w, 0, 0] = sz
        mx[w, 0, 1] = base
        for i, (x, y) in enumerate(chunk):
            mx[w, 0, 2 + i] = x
            my[w, 0, i] = y
    return mx, my

_META_X, _META_Y = _build_meta()


def _bm_body(img_hbm, metax_hbm, metay_hbm, out_hbm, cnts_hbm,
             meta_v, meta_y, imgbuf, rsbuf, nmbuf, cntbuf):
    cid = lax.axis_index("c")
    sid = lax.axis_index("s")
    w = sid * 2 + cid

    @pl.when(w < 24)
    def _copy_through():
        r0 = pl.multiple_of(w * (16 * H), 8)
        pltpu.sync_copy(img_hbm.at[pl.ds(r0, 16 * H)], imgbuf.at[pl.ds(0, 16 * H)])
        pltpu.sync_copy(imgbuf.at[pl.ds(0, 16 * H)], out_hbm.at[pl.ds(r0, 16 * H)])

    pltpu.sync_copy(metax_hbm.at[w], meta_v)
    pltpu.sync_copy(metay_hbm.at[w], meta_y)
    head = meta_v[0, pl.ds(0, 16)]
    qcnt = head[0]
    base = pl.multiple_of(head[1], 8)
    boff = pl.multiple_of(base * H, 8)
    pltpu.sync_copy(img_hbm.at[pl.ds(boff, BWORDS)], imgbuf.at[pl.ds(0, BWORDS)])

    zi16 = jnp.zeros((16,), jnp.int32)
    zf16 = jnp.zeros((16,), jnp.float32)
    lanes = lax.iota(jnp.int32, 16)

    def rs_row(r, _):
        ro = r * H
        for k in range(24):
            co = ro + 16 * k
            sq = [None] * P
            for dj in range(P):
                v = imgbuf[pl.ds(co + dj, 16)]
                sq[dj] = v * v
            s = ((sq[0] + sq[1]) + (sq[2] + sq[3])) + ((sq[4] + sq[5]) + (sq[6] + sq[7]))
            rsbuf[pl.ds(co, 16)] = s
        return 0
    lax.fori_loop(0, BAND, rs_row, 0)

    def nm_row(r, _):
        ro = r * H
        for k in range(24):
            co = ro + 16 * k
            rq = [None] * P
            for di in range(P):
                rq[di] = rsbuf[pl.ds(co + H * di, 16)]
            s = ((rq[0] + rq[1]) + (rq[2] + rq[3])) + ((rq[4] + rq[5]) + (rq[6] + rq[7]))
            nmbuf[pl.ds(co, 16)] = s
        return 0
    lax.fori_loop(0, BAND - P + 1, nm_row, 0)

    @plsc.parallel_loop(0, qcnt, 1)
    def per_query(q):
        x = meta_v[0, pl.ds(2 + q, 16)][0]
        y = meta_y[0, pl.ds(q, 16)][0]
        sx = jnp.maximum(0, x - 20)
        ex = jnp.minimum(H, x + 19)
        sy = jnp.maximum(0, y - 20)
        ey = jnp.minimum(H, y + 19)
        un = ex - P - sx
        vn = ey - P - sy
        sxl = sx - base

        rq = (x - base) * H + y
        thrq = T2 - nmbuf[pl.ds(rq, 16)][0]
        bvals = []
        for di in range(P):
            brow = imgbuf[pl.ds(rq + H * di, 16)]
            for dj in range(P):
                bvals.append(brow[dj])

        @plsc.parallel_loop(0, (un + P - 1) // P, 1, carry=zi16)
        def cvec(ub, cvec):
            u0 = ub * P
            r0 = (sxl + u0) * H + sy
            acc1 = [zf16] * P
            acc2 = [zf16] * P
            for dj in range(P):
                bv = [jnp.full((16,), bvals[di * P + dj], jnp.float32)
                      for di in range(P)]
                for r in range(2 * P - 1):
                    off = r0 + H * r + dj
                    v1 = imgbuf[pl.ds(off, 16)]
                    v2 = imgbuf[pl.ds(off + 16, 16)]
                    for di in range(max(0, r - P + 1), min(P, r + 1)):
                        u = r - di
                        acc1[u] = acc1[u] + v1 * bv[di]
                        acc2[u] = acc2[u] + v2 * bv[di]
            contrib = zi16
            for u in range(P):
                no = (sxl + u0 + u) * H + sy
                n1 = nmbuf[pl.ds(no, 16)]
                n2 = nmbuf[pl.ds(no + 16, 16)]
                m1 = (acc1[u] + acc1[u] > n1 - thrq) & (lanes < vn)
                m2 = (acc2[u] + acc2[u] > n2 - thrq) & (lanes < vn - 16)
                c_u = m1.astype(jnp.int32) + m2.astype(jnp.int32)
                contrib = contrib + jnp.where(u0 + u < un, c_u, zi16)
            return cvec + contrib

        total = jnp.sum(cvec)
        cntbuf[q] = jnp.full((16,), total, jnp.int32)

    pltpu.sync_copy(cntbuf, cnts_hbm.at[w])


def kernel(img):
    metax = jnp.asarray(_META_X)
    metay = jnp.asarray(_META_Y)
    mesh = plsc.VectorSubcoreMesh(core_axis_name="c", subcore_axis_name="s")
    run = functools.partial(
        pl.kernel,
        mesh=mesh,
        compiler_params=pltpu.CompilerParams(needs_layout_passes=False),
        out_type=[
            jax.ShapeDtypeStruct((H * H,), jnp.float32),
            jax.ShapeDtypeStruct((NW, QMAX + 2, 16), jnp.int32),
        ],
        scratch_types=[
            pltpu.VMEM((1, 96), jnp.int32),
            pltpu.VMEM((1, 96), jnp.int32),
            pltpu.VMEM((72 * H,), jnp.float32),
            pltpu.VMEM((72 * H,), jnp.float32),
            pltpu.VMEM((64 * H,), jnp.float32),
            pltpu.VMEM((QMAX + 2, 16), jnp.int32),
        ],
    )(_bm_body)
    out_flat, _counts = run(img.reshape(H * H), metax, metay)
    return out_flat.reshape(H, H)


# speedup vs baseline: 1.3588x; 1.3588x over previous
import functools

import jax
import jax.numpy as jnp
import numpy as np
from jax import lax
from jax.experimental import pallas as pl
from jax.experimental.pallas import tpu as pltpu
from jax.experimental.pallas import tpu_sc as plsc

H = 384
P = 8
NW = 32
QMAX = 70
BAND = 64
BWORDS = BAND * H
T2 = 250.0 * 250.0


def _window(x):
    sx = max(0, x - 20)
    ex = min(H, x + 19)
    return sx, ex


def _build_meta():
    qs = [(x, y) for x in range(0, H - P, P) for y in range(0, H - P, P)]
    n = len(qs)
    sizes = [70] + [69] * 31
    assert sum(sizes) == n
    mx = np.zeros((NW, 1, 96), dtype=np.int32)
    my = np.zeros((NW, 1, 96), dtype=np.int32)
    pos = 0
    for w, sz in enumerate(sizes):
        chunk = qs[pos:pos + sz]
        pos += sz
        sx0, _ = _window(chunk[0][0])
        _, ex1 = _window(chunk[-1][0])
        base = min(8 * (sx0 // 8), H - BAND)
        assert 0 <= base and ex1 - base <= BAND
        mx[w, 0, 0] = sz
        mx[w, 0, 1] = base
        for i, (x, y) in enumerate(chunk):
            mx[w, 0, 2 + i] = x
            my[w, 0, i] = y
    return mx, my

_META_X, _META_Y = _build_meta()


def _bm_body(img_hbm, metax_hbm, metay_hbm, out_hbm, cnts_hbm,
             meta_v, meta_y, imgbuf, rsbuf, nmbuf, cntbuf):
    cid = lax.axis_index("c")
    sid = lax.axis_index("s")
    w = sid * 2 + cid

    @pl.when(w < 24)
    def _copy_through():
        r0 = pl.multiple_of(w * (16 * H), 8)
        pltpu.sync_copy(img_hbm.at[pl.ds(r0, 16 * H)], imgbuf.at[pl.ds(0, 16 * H)])
        pltpu.sync_copy(imgbuf.at[pl.ds(0, 16 * H)], out_hbm.at[pl.ds(r0, 16 * H)])

    pltpu.sync_copy(metax_hbm.at[w], meta_v)
    pltpu.sync_copy(metay_hbm.at[w], meta_y)
    head = meta_v[0, pl.ds(0, 16)]
    qcnt = head[0]
    base = pl.multiple_of(head[1], 8)
    boff = pl.multiple_of(base * H, 8)
    pltpu.sync_copy(img_hbm.at[pl.ds(boff, BWORDS)], imgbuf.at[pl.ds(0, BWORDS)])

    zi16 = jnp.zeros((16,), jnp.int32)
    zf16 = jnp.zeros((16,), jnp.float32)
    lanes = lax.iota(jnp.int32, 16)

    def rs_row(r, _):
        ro = r * H
        for k in range(24):
            co = ro + 16 * k
            sq = [None] * P
            for dj in range(P):
                v = imgbuf[pl.ds(co + dj, 16)]
                sq[dj] = v * v
            s = ((sq[0] + sq[1]) + (sq[2] + sq[3])) + ((sq[4] + sq[5]) + (sq[6] + sq[7]))
            rsbuf[pl.ds(co, 16)] = s
        return 0
    lax.fori_loop(0, BAND, rs_row, 0)

    def nm_row(r, _):
        ro = r * H
        for k in range(24):
            co = ro + 16 * k
            rq = [None] * P
            for di in range(P):
                rq[di] = rsbuf[pl.ds(co + H * di, 16)]
            s = ((rq[0] + rq[1]) + (rq[2] + rq[3])) + ((rq[4] + rq[5]) + (rq[6] + rq[7]))
            nmbuf[pl.ds(co, 16)] = s
        return 0
    lax.fori_loop(0, BAND - P + 1, nm_row, 0)

    @plsc.parallel_loop(0, qcnt, 1)
    def per_query(q):
        x = meta_v[0, pl.ds(2 + q, 16)][0]
        y = meta_y[0, pl.ds(q, 16)][0]
        sx = jnp.maximum(0, x - 20)
        ex = jnp.minimum(H, x + 19)
        sy = jnp.maximum(0, y - 20)
        ey = jnp.minimum(H, y + 19)
        un = ex - P - sx
        vn = ey - P - sy
        sxl = sx - base

        rq = (x - base) * H + y
        thrq = T2 - nmbuf[pl.ds(rq, 16)][0]
        bvals = []
        for di in range(P):
            brow = imgbuf[pl.ds(rq + H * di, 16)]
            for dj in range(P):
                bvals.append(brow[dj])

        zb32 = jnp.zeros((32,), jnp.bfloat16)
        himask = jnp.full((16,), 0xFFFF0000, jnp.uint32)

        @plsc.parallel_loop(0, (un + P - 1) // P, 1, carry=zi16)
        def cvec(ub, cvec):
            u0 = ub * P
            r0 = (sxl + u0) * H + sy
            acc = [zb32] * P
            for dj in range(P):
                bv = []
                for di in range(P):
                    bf = jnp.full((16,), bvals[di * P + dj], jnp.float32)
                    bv.append(plsc.pack(bf, bf, format=plsc.PackFormat.INTERLEAVED))
                for r in range(2 * P - 1):
                    off = r0 + H * r + dj
                    v1 = imgbuf[pl.ds(off, 16)]
                    v2 = imgbuf[pl.ds(off + 16, 16)]
                    vp = plsc.pack(v1, v2, format=plsc.PackFormat.INTERLEAVED)
                    for di in range(max(0, r - P + 1), min(P, r + 1)):
                        u = r - di
                        acc[u] = acc[u] + vp * bv[di]
            contrib = zi16
            for u in range(P):
                no = (sxl + u0 + u) * H + sy
                n1 = nmbuf[pl.ds(no, 16)]
                n2 = nmbuf[pl.ds(no + 16, 16)]
                wbits = plsc.bitcast(acc[u], jnp.uint32)
                a1 = plsc.bitcast(wbits << 16, jnp.float32)
                a2 = plsc.bitcast(wbits & himask, jnp.float32)
                m1 = (a1 + a1 > n1 - thrq) & (lanes < vn)
                m2 = (a2 + a2 > n2 - thrq) & (lanes < vn - 16)
                c_u = m1.astype(jnp.int32) + m2.astype(jnp.int32)
                contrib = contrib + jnp.where(u0 + u < un, c_u, zi16)
            return cvec + contrib

        total = jnp.sum(cvec)
        cntbuf[q] = jnp.full((16,), total, jnp.int32)

    pltpu.sync_copy(cntbuf, cnts_hbm.at[w])


def kernel(img):
    metax = jnp.asarray(_META_X)
    metay = jnp.asarray(_META_Y)
    mesh = plsc.VectorSubcoreMesh(core_axis_name="c", subcore_axis_name="s")
    run = functools.partial(
        pl.kernel,
        mesh=mesh,
        compiler_params=pltpu.CompilerParams(needs_layout_passes=False),
        out_type=[
            jax.ShapeDtypeStruct((H * H,), jnp.float32),
            jax.ShapeDtypeStruct((NW, QMAX + 2, 16), jnp.int32),
        ],
        scratch_types=[
            pltpu.VMEM((1, 96), jnp.int32),
            pltpu.VMEM((1, 96), jnp.int32),
            pltpu.VMEM((72 * H,), jnp.float32),
            pltpu.VMEM((72 * H,), jnp.float32),
            pltpu.VMEM((64 * H,), jnp.float32),
            pltpu.VMEM((QMAX + 2, 16), jnp.int32),
        ],
    )(_bm_body)
    out_flat, _counts = run(img.reshape(H * H), metax, metay)
    return out_flat.reshape(H, H)
